# baseline (speedup 1.0000x reference)
import functools

import jax
import jax.numpy as jnp
from jax import lax
from jax.experimental import pallas as pl
from jax.experimental.pallas import tpu as pltpu
from jax.experimental.pallas import tpu_sc as plsc

_HIDDEN = 64
_GRP = 128
_K = 5
_NC, _NS = 2, 16
_NW = _NC * _NS


@functools.cache
def _make_gather(num_groups):
    gpw = num_groups // _NW
    assert gpw * _NW == num_groups
    nchunks = gpw // _K
    assert nchunks * _K == gpw and nchunks % 2 == 0
    mesh = plsc.VectorSubcoreMesh(core_axis_name="c", subcore_axis_name="s")

    @functools.partial(
        pl.kernel,
        out_type=jax.ShapeDtypeStruct((num_groups, _GRP, _HIDDEN), jnp.float32),
        mesh=mesh,
        scratch_types=[
            pltpu.VMEM((_K * _GRP,), jnp.int32),
            pltpu.VMEM((_K * _GRP,), jnp.int32),
            pltpu.VMEM((_K, _GRP, _HIDDEN), jnp.float32),
            pltpu.VMEM((_K, _GRP, _HIDDEN), jnp.float32),
            pltpu.SemaphoreType.DMA,
            pltpu.SemaphoreType.DMA,
        ],
        compiler_params=pltpu.CompilerParams(use_tc_tiling_on_sc=False),
    )
    def gather(table_hbm, idx_hbm, out_hbm, idx0, idx1, rows0, rows1, sem0, sem1):
        wid = lax.axis_index("s") * _NC + lax.axis_index("c")
        gbase = wid * gpw
        idx_v = (idx0, idx1)
        rows_v = (rows0, rows1)
        sems = (sem0, sem1)

        def fire(c, b):
            g0 = gbase + c * _K
            pltpu.sync_copy(idx_hbm.at[pl.ds(g0 * _GRP, _K * _GRP)], idx_v[b])
            for j in range(_K):
                pltpu.async_copy(
                    table_hbm.at[idx_v[b].at[pl.ds(j * _GRP, _GRP)]],
                    rows_v[b].at[j],
                    sems[b],
                )

        def drain_store(c, b):
            g0 = gbase + c * _K
            pltpu.make_async_copy(
                out_hbm.at[pl.ds(g0, _K)], rows_v[b], sems[b]
            ).wait()
            pltpu.sync_copy(rows_v[b], out_hbm.at[pl.ds(g0, _K)])

        fire(0, 0)

        @pl.loop(0, nchunks, step=2)
        def _(c):
            fire(c + 1, 1)
            drain_store(c, 0)

            @pl.when(c + 2 < nchunks)
            def _():
                fire(c + 2, 0)

            drain_store(c + 1, 1)

    return gather


def _ws_body(ws_ref, out_ref):
    out_ref[...] = jnp.broadcast_to(ws_ref[...], out_ref.shape)


@functools.cache
def _make_ws_broadcast(bs, w):
    blk = 256
    assert bs % blk == 0
    return pl.pallas_call(
        _ws_body,
        grid=(bs // blk,),
        in_specs=[pl.BlockSpec((1, w, _HIDDEN), lambda i: (0, 0, 0))],
        out_specs=pl.BlockSpec((blk, w, _HIDDEN), lambda i: (i, 0, 0)),
        out_shape=jax.ShapeDtypeStruct((bs, w, _HIDDEN), jnp.float32),
    )


def kernel(input_ids, attention_mask, init_workspace, emb_table):
    bs, seq = input_ids.shape
    n = bs * seq
    idx = input_ids.reshape(n)
    emb = _make_gather(n // _GRP)(emb_table, idx)
    embeddings = emb.reshape(bs, seq, _HIDDEN)
    workspace = _make_ws_broadcast(bs, init_workspace.shape[1])(init_workspace)
    return (workspace, embeddings)
